# baseline (speedup 1.0000x reference)
import functools

import jax
import jax.numpy as jnp
from jax import lax
from jax.experimental import pallas as pl
from jax.experimental.pallas import tpu as pltpu
from jax.experimental.pallas import tpu_sc as plsc

N = 10000
E = 320000
D_IN = 128
D_EDGE = 16
D_OUT = 16
D_HID = 32
N_STEPS = 2

NC = 2
NS = 16
NW = NC * NS

EB = 2000
EPW = E // NW
CH = 2000
EPC = E // NC

_PREC = lax.Precision.HIGHEST


def _h0_body(nf_ref, w_ref, b_ref, o_ref):
    o_ref[...] = jnp.maximum(
        jnp.dot(nf_ref[...], w_ref[...], precision=_PREC) + b_ref[...], 0.0)


def _project_nodes(node_feats, proj_W, proj_b):
    return pl.pallas_call(
        _h0_body,
        out_shape=jax.ShapeDtypeStruct((N, D_OUT), jnp.float32),
    )(node_feats, proj_W, proj_b.reshape(1, D_OUT))


def _msg_body(ef_ref, hs_ref, e1w_ref, e1b_ref, w2r_ref, bb_ref, m_ref):
    z = jnp.maximum(
        jnp.dot(ef_ref[...], e1w_ref[...], precision=_PREC) + e1b_ref[...], 0.0)
    h = hs_ref[...]
    p = (z[:, :, None] * h[:, None, :]).reshape(EB, D_HID * D_OUT)
    m_ref[...] = (jnp.dot(p, w2r_ref[...], precision=_PREC)
                  + jnp.dot(h, bb_ref[...], precision=_PREC))


def _messages(edge_feats, h_src, e1_W, e1_b, W2r, Bb):
    grid = (E // EB,)
    return pl.pallas_call(
        _msg_body,
        grid=grid,
        in_specs=[
            pl.BlockSpec((EB, D_EDGE), lambda i: (i, 0)),
            pl.BlockSpec((EB, D_OUT), lambda i: (i, 0)),
            pl.BlockSpec((D_EDGE, D_HID), lambda i: (0, 0)),
            pl.BlockSpec((1, D_HID), lambda i: (0, 0)),
            pl.BlockSpec((D_HID * D_OUT, D_OUT), lambda i: (0, 0)),
            pl.BlockSpec((D_OUT, D_OUT), lambda i: (0, 0)),
        ],
        out_specs=pl.BlockSpec((EB, D_OUT), lambda i: (i, 0)),
        out_shape=jax.ShapeDtypeStruct((E, D_OUT), jnp.float32),
    )(edge_feats, h_src, e1_W, e1_b.reshape(1, D_HID), W2r, Bb)


def _combine_body(p_ref, b_ref, o_ref):
    o_ref[...] = jnp.maximum(p_ref[0] + p_ref[1] + b_ref[...], 0.0)


def _combine(partials, conv_b):
    return pl.pallas_call(
        _combine_body,
        out_shape=jax.ShapeDtypeStruct((N, D_OUT), jnp.float32),
    )(partials, conv_b.reshape(1, D_OUT))


def _final_body(p_ref, b_ref, pw_ref, pb_ref, o_ref):
    h = jnp.maximum(p_ref[0] + p_ref[1] + b_ref[...], 0.0)
    g = jnp.mean(h, axis=0, keepdims=True)
    o_ref[...] = jnp.dot(g, pw_ref[...], precision=_PREC) + pb_ref[...]


def _final(partials, conv_b, pred_W, pred_b):
    return pl.pallas_call(
        _final_body,
        out_shape=jax.ShapeDtypeStruct((1, pred_W.shape[1]), jnp.float32),
    )(partials, conv_b.reshape(1, D_OUT), pred_W,
      pred_b.reshape(1, pred_W.shape[1]))


_MESH = dict(core_axis_name="c", subcore_axis_name="s", num_cores=NC,
             num_subcores=NS)
_SC_PARAMS = pltpu.CompilerParams(use_tc_tiling_on_sc=False)


def _gather_kernel_body(h_hbm, src_hbm, out_hbm, idx_v, rows_v, sem):
    wid = lax.axis_index("s") * NC + lax.axis_index("c")

    def body(i, carry):
        base = wid * EPW + i * CH
        pltpu.sync_copy(src_hbm.at[pl.ds(base, CH)], idx_v)
        pltpu.async_copy(h_hbm.at[idx_v], rows_v, sem).wait()
        pltpu.sync_copy(rows_v, out_hbm.at[pl.ds(base, CH)])
        return carry

    lax.fori_loop(0, EPW // CH, body, 0)


def _sc_gather(h, src):
    k = functools.partial(
        pl.kernel,
        out_type=jax.ShapeDtypeStruct((E, D_OUT), jnp.float32),
        mesh=plsc.VectorSubcoreMesh(**_MESH),
        compiler_params=_SC_PARAMS,
        scratch_types=[
            pltpu.VMEM((CH,), jnp.int32),
            pltpu.VMEM((CH, D_OUT), jnp.float32),
            pltpu.SemaphoreType.DMA,
        ],
    )(_gather_kernel_body)
    return k(h, src)


def _scatter_kernel_body(m_hbm, dst_hbm, zero_hbm, out_hbm, idx_v, rows_v,
                         acc_sh, sem):
    cid = lax.axis_index("c")
    sid = lax.axis_index("s")

    @pl.when(sid == 0)
    def _():
        pltpu.sync_copy(zero_hbm, acc_sh)

    plsc.subcore_barrier()

    def body(i, carry):
        base = cid * EPC + sid * EPW + i * CH
        pltpu.sync_copy(dst_hbm.at[pl.ds(base, CH)], idx_v)
        pltpu.sync_copy(m_hbm.at[pl.ds(base, CH)], rows_v)
        pltpu.sync_copy(rows_v, acc_sh.at[idx_v], add=True)
        return carry

    lax.fori_loop(0, EPW // CH, body, 0)

    plsc.subcore_barrier()

    rows = N // NS
    pltpu.sync_copy(acc_sh.at[pl.ds(sid * rows, rows)],
                    out_hbm.at[cid].at[pl.ds(sid * rows, rows)])


def _sc_scatter(m, dst):
    zeros = jnp.zeros((N, D_OUT), jnp.float32)
    k = functools.partial(
        pl.kernel,
        out_type=jax.ShapeDtypeStruct((NC, N, D_OUT), jnp.float32),
        mesh=plsc.VectorSubcoreMesh(**_MESH),
        compiler_params=_SC_PARAMS,
        scratch_types=[
            pltpu.VMEM((CH,), jnp.int32),
            pltpu.VMEM((CH, D_OUT), jnp.float32),
            pltpu.VMEM_SHARED((N, D_OUT), jnp.float32),
            pltpu.SemaphoreType.DMA,
        ],
    )(_scatter_kernel_body)
    return k(m, dst, zeros)


def kernel(node_feats, edge_feats, edge_index, proj_W, proj_b, e1_W, e1_b,
           e2_W, e2_b, conv_b, pred_W, pred_b):
    src = edge_index[0]
    dst = edge_index[1]
    W2r = e2_W.reshape(D_HID, D_OUT, D_OUT).reshape(D_HID * D_OUT, D_OUT)
    Bb = e2_b.reshape(D_OUT, D_OUT)

    h = _project_nodes(node_feats, proj_W, proj_b)
    for step in range(N_STEPS):
        h_src = _sc_gather(h, src)
        m = _messages(edge_feats, h_src, e1_W, e1_b, W2r, Bb)
        partials = _sc_scatter(m, dst)
        if step < N_STEPS - 1:
            h = _combine(partials, conv_b)
        else:
            out = _final(partials, conv_b, pred_W, pred_b)
    return out


# speedup vs baseline: 7.2611x; 7.2611x over previous
import functools

import jax
import jax.numpy as jnp
from jax import lax
from jax.experimental import pallas as pl
from jax.experimental.pallas import tpu as pltpu
from jax.experimental.pallas import tpu_sc as plsc

N = 10000
E = 320000
D_IN = 128
D_EDGE = 16
D_OUT = 16
D_HID = 32
N_STEPS = 2

NC = 2
NS = 16
NW = NC * NS

EB = 2000
EPW = E // NW
CH = 2000
EPC = E // NC

_PREC = lax.Precision.HIGHEST


def _h0_body(nf_ref, w_ref, b_ref, o_ref):
    o_ref[...] = jnp.maximum(
        jnp.dot(nf_ref[...], w_ref[...], precision=_PREC) + b_ref[...], 0.0)


def _project_nodes(node_feats, proj_W, proj_b):
    return pl.pallas_call(
        _h0_body,
        out_shape=jax.ShapeDtypeStruct((N, D_OUT), jnp.float32),
    )(node_feats, proj_W, proj_b.reshape(1, D_OUT))


PK = 8
PB = 400
KW = PK * D_HID * D_OUT


def _msg_body(ef_ref, hs_ref, e1r_ref, e1b_ref, tile_ref, w2r_ref, bb_ref,
              m_ref):
    ef = ef_ref[...].astype(jnp.bfloat16)
    h = hs_ref[...].astype(jnp.bfloat16)
    z_rep = jnp.maximum(
        jnp.dot(ef, e1r_ref[...], preferred_element_type=jnp.float32)
        + e1b_ref[...], 0.0)
    h_tile = jnp.dot(h, tile_ref[...], preferred_element_type=jnp.float32)
    p = (z_rep * h_tile).astype(jnp.bfloat16)
    m_ref[...] = (
        jnp.dot(p, w2r_ref[...], preferred_element_type=jnp.float32)
        + jnp.dot(h, bb_ref[...], preferred_element_type=jnp.float32))


def _messages(efP, hP, E1R_bd, e1b_bd, TILE_bd, W2r_bd, Bb_bd):
    grid = (E // PK // PB,)
    return pl.pallas_call(
        _msg_body,
        grid=grid,
        in_specs=[
            pl.BlockSpec((PB, PK * D_EDGE), lambda i: (i, 0)),
            pl.BlockSpec((PB, PK * D_OUT), lambda i: (i, 0)),
            pl.BlockSpec((PK * D_EDGE, KW), lambda i: (0, 0)),
            pl.BlockSpec((1, KW), lambda i: (0, 0)),
            pl.BlockSpec((PK * D_OUT, KW), lambda i: (0, 0)),
            pl.BlockSpec((KW, PK * D_OUT), lambda i: (0, 0)),
            pl.BlockSpec((PK * D_OUT, PK * D_OUT), lambda i: (0, 0)),
        ],
        out_specs=pl.BlockSpec((PB, PK * D_OUT), lambda i: (i, 0)),
        out_shape=jax.ShapeDtypeStruct((E // PK, PK * D_OUT), jnp.float32),
    )(efP, hP, E1R_bd, e1b_bd, TILE_bd, W2r_bd, Bb_bd)


def _combine_body(p_ref, b_ref, o_ref):
    o_ref[...] = jnp.maximum(p_ref[0] + p_ref[1] + b_ref[...], 0.0)


def _combine(partials, conv_b):
    return pl.pallas_call(
        _combine_body,
        out_shape=jax.ShapeDtypeStruct((N, D_OUT), jnp.float32),
    )(partials, conv_b.reshape(1, D_OUT))


def _final_body(p_ref, b_ref, pw_ref, pb_ref, o_ref):
    h = jnp.maximum(p_ref[0] + p_ref[1] + b_ref[...], 0.0)
    g = jnp.mean(h, axis=0, keepdims=True)
    o_ref[...] = jnp.dot(g, pw_ref[...], precision=_PREC) + pb_ref[...]


def _final(partials, conv_b, pred_W, pred_b):
    return pl.pallas_call(
        _final_body,
        out_shape=jax.ShapeDtypeStruct((1, pred_W.shape[1]), jnp.float32),
    )(partials, conv_b.reshape(1, D_OUT), pred_W,
      pred_b.reshape(1, pred_W.shape[1]))


_MESH = dict(core_axis_name="c", subcore_axis_name="s", num_cores=NC,
             num_subcores=NS)
_SC_PARAMS = pltpu.CompilerParams(use_tc_tiling_on_sc=False)


def _gather_kernel_body(h_hbm, src_hbm, out_hbm, idx_v, rows_v, sem):
    wid = lax.axis_index("s") * NC + lax.axis_index("c")

    def body(i, carry):
        base = wid * EPW + i * CH
        pltpu.sync_copy(src_hbm.at[pl.ds(base, CH)], idx_v)
        pltpu.async_copy(h_hbm.at[idx_v], rows_v, sem).wait()
        pltpu.sync_copy(rows_v, out_hbm.at[pl.ds(base, CH)])
        return carry

    lax.fori_loop(0, EPW // CH, body, 0)


def _sc_gather(h, src):
    k = functools.partial(
        pl.kernel,
        out_type=jax.ShapeDtypeStruct((E, D_OUT), jnp.float32),
        mesh=plsc.VectorSubcoreMesh(**_MESH),
        compiler_params=_SC_PARAMS,
        scratch_types=[
            pltpu.VMEM((CH,), jnp.int32),
            pltpu.VMEM((CH, D_OUT), jnp.float32),
            pltpu.SemaphoreType.DMA,
        ],
    )(_gather_kernel_body)
    return k(h, src)


def _scatter_kernel_body(m_hbm, dst_hbm, zero_hbm, out_hbm, idx_v, rows_v,
                         acc_sh, sem):
    cid = lax.axis_index("c")
    sid = lax.axis_index("s")

    @pl.when(sid == 0)
    def _():
        pltpu.sync_copy(zero_hbm, acc_sh)

    plsc.subcore_barrier()

    def body(i, carry):
        base = cid * EPC + sid * EPW + i * CH
        pltpu.sync_copy(dst_hbm.at[pl.ds(base, CH)], idx_v)
        pltpu.sync_copy(m_hbm.at[pl.ds(base, CH)], rows_v)
        pltpu.sync_copy(rows_v, acc_sh.at[idx_v], add=True)
        return carry

    lax.fori_loop(0, EPW // CH, body, 0)

    plsc.subcore_barrier()

    rows = N // NS
    pltpu.sync_copy(acc_sh.at[pl.ds(sid * rows, rows)],
                    out_hbm.at[cid].at[pl.ds(sid * rows, rows)])


def _sc_scatter(m, dst):
    zeros = jnp.zeros((N, D_OUT), jnp.float32)
    k = functools.partial(
        pl.kernel,
        out_type=jax.ShapeDtypeStruct((NC, N, D_OUT), jnp.float32),
        mesh=plsc.VectorSubcoreMesh(**_MESH),
        compiler_params=_SC_PARAMS,
        scratch_types=[
            pltpu.VMEM((CH,), jnp.int32),
            pltpu.VMEM((CH, D_OUT), jnp.float32),
            pltpu.VMEM_SHARED((N, D_OUT), jnp.float32),
            pltpu.SemaphoreType.DMA,
        ],
    )(_scatter_kernel_body)
    return k(m, dst, zeros)


def kernel(node_feats, edge_feats, edge_index, proj_W, proj_b, e1_W, e1_b,
           e2_W, e2_b, conv_b, pred_W, pred_b):
    src = edge_index[0]
    dst = edge_index[1]
    f32 = jnp.float32
    bf16 = jnp.bfloat16
    W2r = e2_W.reshape(D_HID, D_OUT, D_OUT).reshape(D_HID * D_OUT, D_OUT)
    Bb = e2_b.reshape(D_OUT, D_OUT)
    eyePK = jnp.eye(PK, dtype=f32)
    REP = jnp.kron(jnp.eye(D_HID, dtype=f32), jnp.ones((1, D_OUT), f32))
    TILE = jnp.tile(jnp.eye(D_OUT, dtype=f32), (1, D_HID))
    E1R_bd = jnp.kron(eyePK, e1_W @ REP).astype(bf16)
    e1b_bd = jnp.tile(e1_b @ REP, PK).reshape(1, KW)
    TILE_bd = jnp.kron(eyePK, TILE).astype(bf16)
    W2r_bd = jnp.kron(eyePK, W2r).astype(bf16)
    Bb_bd = jnp.kron(eyePK, Bb).astype(bf16)
    efP = edge_feats.reshape(E // PK, PK * D_EDGE)

    h = _project_nodes(node_feats, proj_W, proj_b)
    for step in range(N_STEPS):
        h_src = _sc_gather(h, src)
        hP = h_src.reshape(E // PK, PK * D_OUT)
        mP = _messages(efP, hP, E1R_bd, e1b_bd, TILE_bd, W2r_bd, Bb_bd)
        partials = _sc_scatter(mP.reshape(E, D_OUT), dst)
        if step < N_STEPS - 1:
            h = _combine(partials, conv_b)
        else:
            out = _final(partials, conv_b, pred_W, pred_b)
    return out


# speedup vs baseline: 9.8857x; 1.3615x over previous
import functools

import jax
import jax.numpy as jnp
from jax import lax
from jax.experimental import pallas as pl
from jax.experimental.pallas import tpu as pltpu
from jax.experimental.pallas import tpu_sc as plsc

N = 10000
E = 320000
D_IN = 128
D_EDGE = 16
D_OUT = 16
D_HID = 32
N_STEPS = 2

NC = 2
NS = 16
NW = NC * NS

EB = 2000
EPW = E // NW
CH = 2000
EPC = E // NC

_PREC = lax.Precision.HIGHEST


def _h0_body(nf_ref, w_ref, b_ref, o_ref):
    o_ref[...] = jnp.maximum(
        jnp.dot(nf_ref[...], w_ref[...], precision=_PREC) + b_ref[...], 0.0)


def _project_nodes(node_feats, proj_W, proj_b):
    return pl.pallas_call(
        _h0_body,
        out_shape=jax.ShapeDtypeStruct((N, D_OUT), jnp.float32),
    )(node_feats, proj_W, proj_b.reshape(1, D_OUT))


PK = 8
PB = 400
KW = PK * D_HID * D_OUT


def _msg_body(ef_ref, hs_ref, e1x_ref, e1b_ref, w2r_ref, bb_ref, m_ref):
    ef = ef_ref[...].astype(jnp.bfloat16)
    h = hs_ref[...]
    z_kexp = jnp.maximum(
        jnp.dot(ef, e1x_ref[...], preferred_element_type=jnp.float32)
        + e1b_ref[...], 0.0)
    h_tile = jnp.broadcast_to(h[:, None, :], (PB, D_HID, PK * D_OUT)
                              ).reshape(PB, KW)
    p = (z_kexp * h_tile).astype(jnp.bfloat16)
    m_ref[...] = (
        jnp.dot(p, w2r_ref[...], preferred_element_type=jnp.float32)
        + jnp.dot(h.astype(jnp.bfloat16), bb_ref[...],
                  preferred_element_type=jnp.float32))


def _messages(efP, hP, E1X, e1bX, W2rX, Bb_bd):
    grid = (E // PK // PB,)
    return pl.pallas_call(
        _msg_body,
        grid=grid,
        in_specs=[
            pl.BlockSpec((PB, PK * D_EDGE), lambda i: (i, 0)),
            pl.BlockSpec((PB, PK * D_OUT), lambda i: (i, 0)),
            pl.BlockSpec((PK * D_EDGE, KW), lambda i: (0, 0)),
            pl.BlockSpec((1, KW), lambda i: (0, 0)),
            pl.BlockSpec((KW, PK * D_OUT), lambda i: (0, 0)),
            pl.BlockSpec((PK * D_OUT, PK * D_OUT), lambda i: (0, 0)),
        ],
        out_specs=pl.BlockSpec((PB, PK * D_OUT), lambda i: (i, 0)),
        out_shape=jax.ShapeDtypeStruct((E // PK, PK * D_OUT), jnp.float32),
    )(efP, hP, E1X, e1bX, W2rX, Bb_bd)


def _combine_body(p_ref, b_ref, o_ref):
    o_ref[...] = jnp.maximum(p_ref[0] + p_ref[1] + b_ref[...], 0.0)


def _combine(partials, conv_b):
    return pl.pallas_call(
        _combine_body,
        out_shape=jax.ShapeDtypeStruct((N, D_OUT), jnp.float32),
    )(partials, conv_b.reshape(1, D_OUT))


def _final_body(p_ref, b_ref, pw_ref, pb_ref, o_ref):
    h = jnp.maximum(p_ref[0] + p_ref[1] + b_ref[...], 0.0)
    g = jnp.mean(h, axis=0, keepdims=True)
    o_ref[...] = jnp.dot(g, pw_ref[...], precision=_PREC) + pb_ref[...]


def _final(partials, conv_b, pred_W, pred_b):
    return pl.pallas_call(
        _final_body,
        out_shape=jax.ShapeDtypeStruct((1, pred_W.shape[1]), jnp.float32),
    )(partials, conv_b.reshape(1, D_OUT), pred_W,
      pred_b.reshape(1, pred_W.shape[1]))


_MESH = dict(core_axis_name="c", subcore_axis_name="s", num_cores=NC,
             num_subcores=NS)
_SC_PARAMS = pltpu.CompilerParams(use_tc_tiling_on_sc=False)


def _gather_kernel_body(h_hbm, src_hbm, out_hbm, idx_v, rows_v, sem):
    wid = lax.axis_index("s") * NC + lax.axis_index("c")

    def body(i, carry):
        base = wid * EPW + i * CH
        pltpu.sync_copy(src_hbm.at[pl.ds(base, CH)], idx_v)
        pltpu.async_copy(h_hbm.at[idx_v], rows_v, sem).wait()
        pltpu.sync_copy(rows_v, out_hbm.at[pl.ds(base, CH)])
        return carry

    lax.fori_loop(0, EPW // CH, body, 0)


def _sc_gather(h, src):
    k = functools.partial(
        pl.kernel,
        out_type=jax.ShapeDtypeStruct((E, D_OUT), jnp.float32),
        mesh=plsc.VectorSubcoreMesh(**_MESH),
        compiler_params=_SC_PARAMS,
        scratch_types=[
            pltpu.VMEM((CH,), jnp.int32),
            pltpu.VMEM((CH, D_OUT), jnp.float32),
            pltpu.SemaphoreType.DMA,
        ],
    )(_gather_kernel_body)
    return k(h, src)


def _scatter_kernel_body(m_hbm, dst_hbm, zero_hbm, out_hbm, idx_v, rows_v,
                         acc_sh, sem):
    cid = lax.axis_index("c")
    sid = lax.axis_index("s")

    @pl.when(sid == 0)
    def _():
        pltpu.sync_copy(zero_hbm, acc_sh)

    plsc.subcore_barrier()

    def body(i, carry):
        base = cid * EPC + sid * EPW + i * CH
        pltpu.sync_copy(dst_hbm.at[pl.ds(base, CH)], idx_v)
        pltpu.sync_copy(m_hbm.at[pl.ds(base, CH)], rows_v)
        pltpu.sync_copy(rows_v, acc_sh.at[idx_v], add=True)
        return carry

    lax.fori_loop(0, EPW // CH, body, 0)

    plsc.subcore_barrier()

    rows = N // NS
    pltpu.sync_copy(acc_sh.at[pl.ds(sid * rows, rows)],
                    out_hbm.at[cid].at[pl.ds(sid * rows, rows)])


def _sc_scatter(m, dst):
    zeros = jnp.zeros((N, D_OUT), jnp.float32)
    k = functools.partial(
        pl.kernel,
        out_type=jax.ShapeDtypeStruct((NC, N, D_OUT), jnp.float32),
        mesh=plsc.VectorSubcoreMesh(**_MESH),
        compiler_params=_SC_PARAMS,
        scratch_types=[
            pltpu.VMEM((CH,), jnp.int32),
            pltpu.VMEM((CH, D_OUT), jnp.float32),
            pltpu.VMEM_SHARED((N, D_OUT), jnp.float32),
            pltpu.SemaphoreType.DMA,
        ],
    )(_scatter_kernel_body)
    return k(m, dst, zeros)


def kernel(node_feats, edge_feats, edge_index, proj_W, proj_b, e1_W, e1_b,
           e2_W, e2_b, conv_b, pred_W, pred_b):
    src = edge_index[0]
    dst = edge_index[1]
    f32 = jnp.float32
    bf16 = jnp.bfloat16
    W2r = e2_W.reshape(D_HID, D_OUT, D_OUT).reshape(D_HID * D_OUT, D_OUT)
    Bb = e2_b.reshape(D_OUT, D_OUT)
    eyePK = jnp.eye(PK, dtype=f32)
    ones16 = jnp.ones((D_OUT,), f32)
    E1X = jnp.einsum('ck,jJ,i->jckJi', e1_W, eyePK, ones16
                     ).reshape(PK * D_EDGE, KW).astype(bf16)
    e1bX = jnp.einsum('k,j,i->kji', e1_b, jnp.ones((PK,), f32), ones16
                      ).reshape(1, KW)
    W2rX = jnp.einsum('kio,jJ->kjiJo',
                      e2_W.reshape(D_HID, D_OUT, D_OUT), eyePK
                      ).reshape(KW, PK * D_OUT).astype(bf16)
    Bb_bd = jnp.kron(eyePK, Bb).astype(bf16)
    efP = edge_feats.reshape(E // PK, PK * D_EDGE)

    h = _project_nodes(node_feats, proj_W, proj_b)
    for step in range(N_STEPS):
        h_src = _sc_gather(h, src)
        hP = h_src.reshape(E // PK, PK * D_OUT)
        mP = _messages(efP, hP, E1X, e1bX, W2rX, Bb_bd)
        partials = _sc_scatter(mP.reshape(E, D_OUT), dst)
        if step < N_STEPS - 1:
            h = _combine(partials, conv_b)
        else:
            out = _final(partials, conv_b, pred_W, pred_b)
    return out


# speedup vs baseline: 14.0298x; 1.4192x over previous
import functools

import jax
import jax.numpy as jnp
from jax import lax
from jax.experimental import pallas as pl
from jax.experimental.pallas import tpu as pltpu
from jax.experimental.pallas import tpu_sc as plsc

N = 10000
E = 320000
D_IN = 128
D_EDGE = 16
D_OUT = 16
D_HID = 32
N_STEPS = 2

NC = 2
NS = 16
NW = NC * NS

EB = 2000
EPW = E // NW
CH = 2000
EPC = E // NC

_PREC = lax.Precision.HIGHEST


def _h0_body(nf_ref, w_ref, b_ref, o_ref):
    o_ref[...] = jnp.maximum(
        jnp.dot(nf_ref[...], w_ref[...], precision=_PREC) + b_ref[...], 0.0)


def _project_nodes(node_feats, proj_W, proj_b):
    return pl.pallas_call(
        _h0_body,
        out_shape=jax.ShapeDtypeStruct((N, D_OUT), jnp.float32),
    )(node_feats, proj_W, proj_b.reshape(1, D_OUT))


PK = 8
PB = 1000
KW = PK * D_HID * D_OUT


def _msg_body(ef_ref, hs_ref, e1wt_ref, e1bt_ref, w2rt_ref, bbt_ref, m_ref):
    efT = ef_ref[...].T.astype(jnp.bfloat16)
    hT = hs_ref[...].T
    e1wt = e1wt_ref[...]
    e1bt = e1bt_ref[...]
    w2rt = w2rt_ref[...]
    bbt = bbt_ref[...]
    parts = []
    for j in range(PK):
        efj = efT[j * D_EDGE:(j + 1) * D_EDGE, :]
        hj = hT[j * D_OUT:(j + 1) * D_OUT, :]
        zj = jnp.maximum(
            jnp.dot(e1wt, efj, preferred_element_type=jnp.float32) + e1bt,
            0.0)
        pj = (zj[:, None, :] * hj[None, :, :]).reshape(
            D_HID * D_OUT, PB).astype(jnp.bfloat16)
        mj = (jnp.dot(w2rt, pj, preferred_element_type=jnp.float32)
              + jnp.dot(bbt, hj.astype(jnp.bfloat16),
                        preferred_element_type=jnp.float32))
        parts.append(mj)
    mT = jnp.concatenate(parts, axis=0)
    m_ref[...] = mT.T


def _messages(efP, hP, e1_WT, e1_bT, W2rT, BbT):
    grid = (E // PK // PB,)
    return pl.pallas_call(
        _msg_body,
        grid=grid,
        in_specs=[
            pl.BlockSpec((PB, PK * D_EDGE), lambda i: (i, 0)),
            pl.BlockSpec((PB, PK * D_OUT), lambda i: (i, 0)),
            pl.BlockSpec((D_HID, D_EDGE), lambda i: (0, 0)),
            pl.BlockSpec((D_HID, 1), lambda i: (0, 0)),
            pl.BlockSpec((D_OUT, D_HID * D_OUT), lambda i: (0, 0)),
            pl.BlockSpec((D_OUT, D_OUT), lambda i: (0, 0)),
        ],
        out_specs=pl.BlockSpec((PB, PK * D_OUT), lambda i: (i, 0)),
        out_shape=jax.ShapeDtypeStruct((E // PK, PK * D_OUT), jnp.float32),
    )(efP, hP, e1_WT, e1_bT, W2rT, BbT)


def _combine_body(p_ref, b_ref, o_ref):
    o_ref[...] = jnp.maximum(p_ref[0] + p_ref[1] + b_ref[...], 0.0)


def _combine(partials, conv_b):
    return pl.pallas_call(
        _combine_body,
        out_shape=jax.ShapeDtypeStruct((N, D_OUT), jnp.float32),
    )(partials, conv_b.reshape(1, D_OUT))


def _final_body(p_ref, b_ref, pw_ref, pb_ref, o_ref):
    h = jnp.maximum(p_ref[0] + p_ref[1] + b_ref[...], 0.0)
    g = jnp.mean(h, axis=0, keepdims=True)
    o_ref[...] = jnp.dot(g, pw_ref[...], precision=_PREC) + pb_ref[...]


def _final(partials, conv_b, pred_W, pred_b):
    return pl.pallas_call(
        _final_body,
        out_shape=jax.ShapeDtypeStruct((1, pred_W.shape[1]), jnp.float32),
    )(partials, conv_b.reshape(1, D_OUT), pred_W,
      pred_b.reshape(1, pred_W.shape[1]))


_MESH = dict(core_axis_name="c", subcore_axis_name="s", num_cores=NC,
             num_subcores=NS)
_SC_PARAMS = pltpu.CompilerParams(use_tc_tiling_on_sc=False)


def _gather_kernel_body(h_hbm, src_hbm, out_hbm, idx_v, rows_v, sem):
    wid = lax.axis_index("s") * NC + lax.axis_index("c")

    def body(i, carry):
        base = wid * EPW + i * CH
        pltpu.sync_copy(src_hbm.at[pl.ds(base, CH)], idx_v)
        pltpu.async_copy(h_hbm.at[idx_v], rows_v, sem).wait()
        pltpu.sync_copy(rows_v, out_hbm.at[pl.ds(base, CH)])
        return carry

    lax.fori_loop(0, EPW // CH, body, 0)


def _sc_gather(h, src):
    k = functools.partial(
        pl.kernel,
        out_type=jax.ShapeDtypeStruct((E, D_OUT), jnp.float32),
        mesh=plsc.VectorSubcoreMesh(**_MESH),
        compiler_params=_SC_PARAMS,
        scratch_types=[
            pltpu.VMEM((CH,), jnp.int32),
            pltpu.VMEM((CH, D_OUT), jnp.float32),
            pltpu.SemaphoreType.DMA,
        ],
    )(_gather_kernel_body)
    return k(h, src)


def _scatter_kernel_body(m_hbm, dst_hbm, zero_hbm, out_hbm, idx_v, rows_v,
                         acc_sh, sem):
    cid = lax.axis_index("c")
    sid = lax.axis_index("s")

    @pl.when(sid == 0)
    def _():
        pltpu.sync_copy(zero_hbm, acc_sh)

    plsc.subcore_barrier()

    def body(i, carry):
        base = cid * EPC + sid * EPW + i * CH
        pltpu.sync_copy(dst_hbm.at[pl.ds(base, CH)], idx_v)
        pltpu.sync_copy(m_hbm.at[pl.ds(base, CH)], rows_v)
        pltpu.sync_copy(rows_v, acc_sh.at[idx_v], add=True)
        return carry

    lax.fori_loop(0, EPW // CH, body, 0)

    plsc.subcore_barrier()

    rows = N // NS
    pltpu.sync_copy(acc_sh.at[pl.ds(sid * rows, rows)],
                    out_hbm.at[cid].at[pl.ds(sid * rows, rows)])


def _sc_scatter(m, dst):
    zeros = jnp.zeros((N, D_OUT), jnp.float32)
    k = functools.partial(
        pl.kernel,
        out_type=jax.ShapeDtypeStruct((NC, N, D_OUT), jnp.float32),
        mesh=plsc.VectorSubcoreMesh(**_MESH),
        compiler_params=_SC_PARAMS,
        scratch_types=[
            pltpu.VMEM((CH,), jnp.int32),
            pltpu.VMEM((CH, D_OUT), jnp.float32),
            pltpu.VMEM_SHARED((N, D_OUT), jnp.float32),
            pltpu.SemaphoreType.DMA,
        ],
    )(_scatter_kernel_body)
    return k(m, dst, zeros)


def kernel(node_feats, edge_feats, edge_index, proj_W, proj_b, e1_W, e1_b,
           e2_W, e2_b, conv_b, pred_W, pred_b):
    src = edge_index[0]
    dst = edge_index[1]
    f32 = jnp.float32
    bf16 = jnp.bfloat16
    W2r = e2_W.reshape(D_HID, D_OUT, D_OUT).reshape(D_HID * D_OUT, D_OUT)
    Bb = e2_b.reshape(D_OUT, D_OUT)
    e1_WT = e1_W.T.astype(bf16)
    e1_bT = e1_b.reshape(D_HID, 1)
    W2rT = W2r.T.astype(bf16)
    BbT = Bb.T.astype(bf16)
    efP = edge_feats.reshape(E // PK, PK * D_EDGE)

    h = _project_nodes(node_feats, proj_W, proj_b)
    for step in range(N_STEPS):
        h_src = _sc_gather(h, src)
        hP = h_src.reshape(E // PK, PK * D_OUT)
        mP = _messages(efP, hP, e1_WT, e1_bT, W2rT, BbT)
        partials = _sc_scatter(mP.reshape(E, D_OUT), dst)
        if step < N_STEPS - 1:
            h = _combine(partials, conv_b)
        else:
            out = _final(partials, conv_b, pred_W, pred_b)
    return out
